# initial kernel scaffold (unmeasured)
import jax
import jax.numpy as jnp
from jax import lax
from jax.experimental import pallas as pl
from jax.experimental.pallas import tpu as pltpu

N_DEV = 8
SQ = 1024
D = 1024
HQ = 8
DH = 128
CHUNK = SQ // N_DEV
SCALE = 0.08838834764831843


def kernel(x, Wq, Wk, Wv, Wo):
    x2 = x.reshape(SQ, D)

    def body(x_ref, wq_ref, wk_ref, wv_ref, wo_ref, out_ref,
             comm_ref, send_sems, rs_sems, ag_sems):
        my = lax.axis_index("i")
        left = lax.rem(my + N_DEV - 1, N_DEV)
        right = lax.rem(my + 1, N_DEV)

        barrier_sem = pltpu.get_barrier_semaphore()
        for nbr in (left, right):
            pl.semaphore_signal(barrier_sem, inc=1, device_id=(nbr,),
                                device_id_type=pl.DeviceIdType.MESH)
        pl.semaphore_wait(barrier_sem, 2)

        xv = x_ref[...]
        d_idx = lax.broadcasted_iota(jnp.int32, (SQ, DH), 1)
        pos = lax.broadcasted_iota(jnp.float32, (SQ, DH), 0)
        inv = jnp.exp(-jnp.log(10000.0) * ((d_idx // 2) * 2).astype(jnp.float32) / DH)
        ang = pos * inv
        cos_t = jnp.cos(ang)
        sin_t = jnp.sin(ang)
        even = (d_idx % 2) == 0

        def rope(t):
            rot = jnp.where(even, -pltpu.roll(t, -1, 1), pltpu.roll(t, 1, 1))
            return t * cos_t + rot * sin_t

        partial = jnp.zeros((SQ, D), jnp.float32)
        for h in range(HQ):
            sl = slice(h * DH, (h + 1) * DH)
            qh = rope(jnp.dot(xv, wq_ref[:, sl], preferred_element_type=jnp.float32))
            kh = rope(jnp.dot(xv, wk_ref[:, sl], preferred_element_type=jnp.float32))
            vh = jnp.dot(xv, wv_ref[:, sl], preferred_element_type=jnp.float32)
            s = lax.dot_general(qh, kh, (((1,), (1,)), ((), ())),
                                preferred_element_type=jnp.float32) * SCALE
            s = s - jnp.max(s, axis=-1, keepdims=True)
            w = jnp.exp(s)
            w = w / jnp.sum(w, axis=-1, keepdims=True)
            ctx = jnp.dot(w, vh, preferred_element_type=jnp.float32)
            partial = partial + jnp.dot(ctx, wo_ref[sl, :],
                                        preferred_element_type=jnp.float32)
        out_ref[...] = partial

        for hop in range(N_DEV - 1):
            send_c = lax.rem(my + N_DEV - hop, N_DEV)
            recv_c = lax.rem(my + N_DEV - hop - 1, N_DEV)
            rdma = pltpu.make_async_remote_copy(
                src_ref=out_ref.at[pl.ds(send_c * CHUNK, CHUNK), :],
                dst_ref=comm_ref.at[hop],
                send_sem=send_sems.at[hop],
                recv_sem=rs_sems.at[hop],
                device_id=(right,),
                device_id_type=pl.DeviceIdType.MESH,
            )
            rdma.start()
            rdma.wait()
            out_ref[pl.ds(recv_c * CHUNK, CHUNK), :] = (
                out_ref[pl.ds(recv_c * CHUNK, CHUNK), :] + comm_ref[hop]
            )

        for hop in range(N_DEV - 1):
            send_c = lax.rem(my + 1 + N_DEV - hop, N_DEV)
            rdma = pltpu.make_async_remote_copy(
                src_ref=out_ref.at[pl.ds(send_c * CHUNK, CHUNK), :],
                dst_ref=out_ref.at[pl.ds(send_c * CHUNK, CHUNK), :],
                send_sem=send_sems.at[N_DEV - 1 + hop],
                recv_sem=ag_sems.at[hop],
                device_id=(right,),
                device_id_type=pl.DeviceIdType.MESH,
            )
            rdma.start()
            rdma.wait()

    out = pl.pallas_call(
        body,
        out_shape=jax.ShapeDtypeStruct((SQ, D), jnp.float32),
        in_specs=[pl.BlockSpec(memory_space=pltpu.VMEM)] * 5,
        out_specs=pl.BlockSpec(memory_space=pltpu.VMEM),
        scratch_shapes=[
            pltpu.VMEM((N_DEV - 1, CHUNK, D), jnp.float32),
            pltpu.SemaphoreType.DMA((2 * (N_DEV - 1),)),
            pltpu.SemaphoreType.DMA((N_DEV - 1,)),
            pltpu.SemaphoreType.DMA((N_DEV - 1,)),
        ],
        compiler_params=pltpu.CompilerParams(collective_id=0),
    )(x2, Wq, Wk, Wv, Wo)
    return out.reshape(1, SQ, D)


# baseline (device time: 166480 ns/iter reference)
import jax
import jax.numpy as jnp
from jax import lax
from jax.experimental import pallas as pl
from jax.experimental.pallas import tpu as pltpu

N_DEV = 8
SQ = 1024
D = 1024
HQ = 8
DH = 128
CHUNK = SQ // N_DEV
SCALE = 0.08838834764831843


def kernel(x, Wq, Wk, Wv, Wo):
    x2 = x.reshape(SQ, D)

    def body(x_ref, wq_ref, wk_ref, wv_ref, wo_ref, out_ref,
             comm_ref, send_sems, rs_sems, ag_sems):
        my = lax.axis_index("i")
        left = lax.rem(my + N_DEV - 1, N_DEV)
        right = lax.rem(my + 1, N_DEV)

        barrier_sem = pltpu.get_barrier_semaphore()
        for nbr in (left, right):
            pl.semaphore_signal(barrier_sem, inc=1, device_id=(nbr,),
                                device_id_type=pl.DeviceIdType.MESH)
        pl.semaphore_wait(barrier_sem, 2)

        xv = x_ref[...]
        d_idx = lax.broadcasted_iota(jnp.int32, (SQ, DH), 1)
        pos = lax.broadcasted_iota(jnp.int32, (SQ, DH), 0).astype(jnp.float32)
        inv = jnp.exp(-jnp.log(10000.0) * ((d_idx // 2) * 2).astype(jnp.float32) / DH)
        ang = pos * inv
        cos_t = jnp.cos(ang)
        sin_t = jnp.sin(ang)
        even = (d_idx % 2) == 0

        def rope(t):
            rot = jnp.where(even, -pltpu.roll(t, DH - 1, 1), pltpu.roll(t, 1, 1))
            return t * cos_t + rot * sin_t

        partial = jnp.zeros((SQ, D), jnp.float32)
        for h in range(HQ):
            sl = slice(h * DH, (h + 1) * DH)
            qh = rope(jnp.dot(xv, wq_ref[:, sl], preferred_element_type=jnp.float32))
            kh = rope(jnp.dot(xv, wk_ref[:, sl], preferred_element_type=jnp.float32))
            vh = jnp.dot(xv, wv_ref[:, sl], preferred_element_type=jnp.float32)
            s = lax.dot_general(qh, kh, (((1,), (1,)), ((), ())),
                                preferred_element_type=jnp.float32) * SCALE
            s = s - jnp.max(s, axis=-1, keepdims=True)
            w = jnp.exp(s)
            w = w / jnp.sum(w, axis=-1, keepdims=True)
            ctx = jnp.dot(w, vh, preferred_element_type=jnp.float32)
            partial = partial + jnp.dot(ctx, wo_ref[sl, :],
                                        preferred_element_type=jnp.float32)
        out_ref[...] = partial

        for hop in range(N_DEV - 1):
            send_c = lax.rem(my + N_DEV - hop, N_DEV)
            recv_c = lax.rem(my + N_DEV - hop - 1, N_DEV)
            rdma = pltpu.make_async_remote_copy(
                src_ref=out_ref.at[pl.ds(send_c * CHUNK, CHUNK), :],
                dst_ref=comm_ref.at[hop],
                send_sem=send_sems.at[hop],
                recv_sem=rs_sems.at[hop],
                device_id=(right,),
                device_id_type=pl.DeviceIdType.MESH,
            )
            rdma.start()
            rdma.wait()
            out_ref[pl.ds(recv_c * CHUNK, CHUNK), :] = (
                out_ref[pl.ds(recv_c * CHUNK, CHUNK), :] + comm_ref[hop]
            )

        for hop in range(N_DEV - 1):
            send_c = lax.rem(my + 1 + N_DEV - hop, N_DEV)
            rdma = pltpu.make_async_remote_copy(
                src_ref=out_ref.at[pl.ds(send_c * CHUNK, CHUNK), :],
                dst_ref=out_ref.at[pl.ds(send_c * CHUNK, CHUNK), :],
                send_sem=send_sems.at[N_DEV - 1 + hop],
                recv_sem=ag_sems.at[hop],
                device_id=(right,),
                device_id_type=pl.DeviceIdType.MESH,
            )
            rdma.start()
            rdma.wait()

    out = pl.pallas_call(
        body,
        out_shape=jax.ShapeDtypeStruct((SQ, D), jnp.float32),
        in_specs=[pl.BlockSpec(memory_space=pltpu.VMEM)] * 5,
        out_specs=pl.BlockSpec(memory_space=pltpu.VMEM),
        scratch_shapes=[
            pltpu.VMEM((N_DEV - 1, CHUNK, D), jnp.float32),
            pltpu.SemaphoreType.DMA((2 * (N_DEV - 1),)),
            pltpu.SemaphoreType.DMA((N_DEV - 1,)),
            pltpu.SemaphoreType.DMA((N_DEV - 1,)),
        ],
        compiler_params=pltpu.CompilerParams(collective_id=0),
    )(x2, Wq, Wk, Wv, Wo)
    return out.reshape(1, SQ, D)
